# baseline (speedup 1.0000x reference)
import functools

import jax
import jax.numpy as jnp
from jax import lax
from jax.experimental import pallas as pl
from jax.experimental.pallas import tpu as pltpu
from jax.experimental.pallas import tpu_sc as plsc

HID = 128
OUT = 512

_NC = 2
_NS = 16
_NW = _NC * _NS

_K = 64
_NB = 8
_NCHUNK = 2
_BLK = 8192


def _sc_gather(x2d, table):
    n_chunks = x2d.shape[0]
    chunks_per_w = n_chunks // _NW
    rows_per_w = chunks_per_w * _K
    n_groups = chunks_per_w // _NB
    rem = chunks_per_w - n_groups * _NB

    mesh = plsc.VectorSubcoreMesh(core_axis_name="c", subcore_axis_name="s")

    @functools.partial(
        pl.kernel,
        mesh=mesh,
        out_type=jax.ShapeDtypeStruct((n_chunks * _K, HID), jnp.float32),
        scratch_types=[
            pltpu.VMEM((chunks_per_w, _K), jnp.int32),
            pltpu.VMEM((_NB, _K, HID), jnp.float32),
        ] + [pltpu.SemaphoreType.DMA] * _NB,
    )
    def gather_kernel(idx_hbm, table_hbm, h_hbm, idx_v, rows_v, *gsems):
        wid = lax.axis_index("s") * _NC + lax.axis_index("c")
        chunk0 = wid * chunks_per_w
        row0 = wid * rows_per_w

        pltpu.sync_copy(idx_hbm.at[pl.ds(chunk0, chunks_per_w)], idx_v)

        for b in range(_NB):
            pltpu.async_copy(table_hbm.at[idx_v.at[b]], rows_v.at[b], gsems[b])

        def step(j, b):
            pltpu.make_async_copy(
                table_hbm.at[idx_v.at[0]], rows_v.at[b], gsems[b]
            ).wait()
            pltpu.sync_copy(rows_v.at[b], h_hbm.at[pl.ds(row0 + j * _K, _K)])

            @pl.when(j + _NB < chunks_per_w)
            def _():
                pltpu.async_copy(
                    table_hbm.at[idx_v.at[j + _NB]], rows_v.at[b], gsems[b]
                )

        def body(g, _):
            for b in range(_NB):
                step(g * _NB + b, b)
            return 0

        lax.fori_loop(0, n_groups, body, 0)
        for b in range(rem):
            step(n_groups * _NB + b, b)

    return gather_kernel(x2d, table)


def _tc_project_into(h, W_up, b2d, out_prev, n_total, row0):
    n_rows = h.shape[0]
    i0 = row0 // _BLK

    in_specs = [
        pl.BlockSpec((_BLK, HID), lambda j: (j, 0)),
        pl.BlockSpec((HID, OUT), lambda j: (0, 0)),
        pl.BlockSpec((1, OUT), lambda j: (0, 0)),
    ]
    inputs = [h, W_up, b2d]
    kwargs = {}
    if out_prev is not None:
        in_specs.append(pl.BlockSpec(memory_space=pl.ANY))
        inputs.append(out_prev)
        kwargs["input_output_aliases"] = {3: 0}

    def mm_kernel(h_ref, w_ref, b_ref, *rest):
        o_ref = rest[-1]
        hb = jnp.maximum(h_ref[...], 0.0)
        o_ref[...] = (
            jnp.dot(hb, w_ref[...], preferred_element_type=jnp.float32)
            + b_ref[...]
        )

    return pl.pallas_call(
        mm_kernel,
        grid=(n_rows // _BLK,),
        in_specs=in_specs,
        out_specs=pl.BlockSpec((_BLK, OUT), lambda j: (i0 + j, 0)),
        out_shape=jax.ShapeDtypeStruct((n_total, OUT), jnp.float32),
        **kwargs,
    )(*inputs)


def kernel(x, emb_table, W_up, b_up):
    B, L = x.shape
    n = B * L
    x2d = x.reshape(n // _K, _K).astype(jnp.int32)
    b2d = b_up.reshape(1, OUT)

    rows_per_chunk = n // _NCHUNK
    idx_per_chunk = x2d.shape[0] // _NCHUNK

    hs = [
        _sc_gather(
            lax.slice_in_dim(x2d, i * idx_per_chunk, (i + 1) * idx_per_chunk),
            emb_table,
        )
        for i in range(_NCHUNK)
    ]
    out = None
    for i in range(_NCHUNK):
        out = _tc_project_into(hs[i], W_up, b2d, out, n, i * rows_per_chunk)
    return out.reshape(B, L, OUT)


# speedup vs baseline: 1.0060x; 1.0060x over previous
import functools

import jax
import jax.numpy as jnp
from jax import lax
from jax.experimental import pallas as pl
from jax.experimental.pallas import tpu as pltpu
from jax.experimental.pallas import tpu_sc as plsc

HID = 128
OUT = 512

_NC = 2
_NS = 16
_NW = _NC * _NS

_K = 128
_NB = 4
_NCHUNK = 1
_BLK = 8192


def _sc_gather(x2d, table):
    n_chunks = x2d.shape[0]
    chunks_per_w = n_chunks // _NW
    rows_per_w = chunks_per_w * _K
    n_groups = chunks_per_w // _NB
    rem = chunks_per_w - n_groups * _NB

    mesh = plsc.VectorSubcoreMesh(core_axis_name="c", subcore_axis_name="s")

    @functools.partial(
        pl.kernel,
        mesh=mesh,
        out_type=jax.ShapeDtypeStruct((n_chunks * _K, HID), jnp.float32),
        scratch_types=[
            pltpu.VMEM((chunks_per_w, _K), jnp.int32),
            pltpu.VMEM((_NB, _K, HID), jnp.float32),
        ] + [pltpu.SemaphoreType.DMA] * (2 * _NB),
    )
    def gather_kernel(idx_hbm, table_hbm, h_hbm, idx_v, rows_v, *sems):
        gsems = sems[:_NB]
        osems = sems[_NB:]
        wid = lax.axis_index("s") * _NC + lax.axis_index("c")
        chunk0 = wid * chunks_per_w
        row0 = wid * rows_per_w
        half = _NB // 2

        pltpu.sync_copy(idx_hbm.at[pl.ds(chunk0, chunks_per_w)], idx_v)

        def fire_gather(j, b):
            pltpu.async_copy(table_hbm.at[idx_v.at[j]], rows_v.at[b], gsems[b])

        def fire_wb(j, b):
            pltpu.async_copy(
                rows_v.at[b], h_hbm.at[pl.ds(row0 + j * _K, _K)], osems[b]
            )

        def wait_gather(b):
            pltpu.make_async_copy(
                table_hbm.at[idx_v.at[0]], rows_v.at[b], gsems[b]
            ).wait()

        def wait_wb(b):
            pltpu.make_async_copy(
                rows_v.at[b], h_hbm.at[pl.ds(row0, _K)], osems[b]
            ).wait()

        for b in range(half):
            fire_gather(b, b)

        def step(j, b, b2):
            wait_gather(b)
            fire_wb(j, b)

            @pl.when(j + half < chunks_per_w)
            def _():
                @pl.when(j >= half)
                def _():
                    wait_wb(b2)
                fire_gather(j + half, b2)

        def body(g, _):
            for i in range(_NB):
                step(g * _NB + i, i, (i + half) % _NB)
            return 0

        lax.fori_loop(0, n_groups, body, 0)
        for i in range(rem):
            step(n_groups * _NB + i, i, (i + half) % _NB)

        for b in range(min(_NB, chunks_per_w)):
            wait_wb(b)

    return gather_kernel(x2d, table)


def _tc_project_into(h, W_up, b2d, out_prev, n_total, row0):
    n_rows = h.shape[0]
    i0 = row0 // _BLK

    in_specs = [
        pl.BlockSpec((_BLK, HID), lambda j: (j, 0)),
        pl.BlockSpec((HID, OUT), lambda j: (0, 0)),
        pl.BlockSpec((1, OUT), lambda j: (0, 0)),
    ]
    inputs = [h, W_up, b2d]
    kwargs = {}
    if out_prev is not None:
        in_specs.append(pl.BlockSpec(memory_space=pl.ANY))
        inputs.append(out_prev)
        kwargs["input_output_aliases"] = {3: 0}

    def mm_kernel(h_ref, w_ref, b_ref, *rest):
        o_ref = rest[-1]
        hb = jnp.maximum(h_ref[...], 0.0)
        o_ref[...] = (
            jnp.dot(hb, w_ref[...], preferred_element_type=jnp.float32)
            + b_ref[...]
        )

    return pl.pallas_call(
        mm_kernel,
        grid=(n_rows // _BLK,),
        in_specs=in_specs,
        out_specs=pl.BlockSpec((_BLK, OUT), lambda j: (i0 + j, 0)),
        out_shape=jax.ShapeDtypeStruct((n_total, OUT), jnp.float32),
        **kwargs,
    )(*inputs)


def kernel(x, emb_table, W_up, b_up):
    B, L = x.shape
    n = B * L
    x2d = x.reshape(n // _K, _K).astype(jnp.int32)
    b2d = b_up.reshape(1, OUT)

    rows_per_chunk = n // _NCHUNK
    idx_per_chunk = x2d.shape[0] // _NCHUNK

    hs = [
        _sc_gather(
            lax.slice_in_dim(x2d, i * idx_per_chunk, (i + 1) * idx_per_chunk),
            emb_table,
        )
        for i in range(_NCHUNK)
    ]
    out = None
    for i in range(_NCHUNK):
        out = _tc_project_into(hs[i], W_up, b2d, out, n, i * rows_per_chunk)
    return out.reshape(B, L, OUT)


# speedup vs baseline: 1.0109x; 1.0048x over previous
import functools

import jax
import jax.numpy as jnp
from jax import lax
from jax.experimental import pallas as pl
from jax.experimental.pallas import tpu as pltpu
from jax.experimental.pallas import tpu_sc as plsc

HID = 128
OUT = 512

_NC = 2
_NS = 16
_NW = _NC * _NS

_K = 128
_NB = 4
_NCHUNK = 1
_BLK = 10240


def _sc_gather(x2d, table):
    n_chunks = x2d.shape[0]
    chunks_per_w = n_chunks // _NW
    rows_per_w = chunks_per_w * _K
    n_groups = chunks_per_w // _NB
    rem = chunks_per_w - n_groups * _NB

    mesh = plsc.VectorSubcoreMesh(core_axis_name="c", subcore_axis_name="s")

    @functools.partial(
        pl.kernel,
        mesh=mesh,
        out_type=jax.ShapeDtypeStruct((n_chunks * _K, HID), jnp.float32),
        scratch_types=[
            pltpu.VMEM((chunks_per_w, _K), jnp.int32),
            pltpu.VMEM((_NB, _K, HID), jnp.float32),
        ] + [pltpu.SemaphoreType.DMA] * (2 * _NB),
    )
    def gather_kernel(idx_hbm, table_hbm, h_hbm, idx_v, rows_v, *sems):
        gsems = sems[:_NB]
        osems = sems[_NB:]
        wid = lax.axis_index("s") * _NC + lax.axis_index("c")
        chunk0 = wid * chunks_per_w
        row0 = wid * rows_per_w
        half = _NB // 2

        pltpu.sync_copy(idx_hbm.at[pl.ds(chunk0, chunks_per_w)], idx_v)

        def fire_gather(j, b):
            pltpu.async_copy(table_hbm.at[idx_v.at[j]], rows_v.at[b], gsems[b])

        def fire_wb(j, b):
            pltpu.async_copy(
                rows_v.at[b], h_hbm.at[pl.ds(row0 + j * _K, _K)], osems[b]
            )

        def wait_gather(b):
            pltpu.make_async_copy(
                table_hbm.at[idx_v.at[0]], rows_v.at[b], gsems[b]
            ).wait()

        def wait_wb(b):
            pltpu.make_async_copy(
                rows_v.at[b], h_hbm.at[pl.ds(row0, _K)], osems[b]
            ).wait()

        for b in range(half):
            fire_gather(b, b)

        def step(j, b, b2):
            wait_gather(b)
            fire_wb(j, b)

            @pl.when(j + half < chunks_per_w)
            def _():
                @pl.when(j >= half)
                def _():
                    wait_wb(b2)
                fire_gather(j + half, b2)

        def body(g, _):
            for i in range(_NB):
                step(g * _NB + i, i, (i + half) % _NB)
            return 0

        lax.fori_loop(0, n_groups, body, 0)
        for i in range(rem):
            step(n_groups * _NB + i, i, (i + half) % _NB)

        for b in range(min(_NB, chunks_per_w)):
            wait_wb(b)

    return gather_kernel(x2d, table)


def _tc_project_into(h, W_up, b2d, out_prev, n_total, row0):
    n_rows = h.shape[0]
    i0 = row0 // _BLK

    in_specs = [
        pl.BlockSpec((_BLK, HID), lambda j: (j, 0)),
        pl.BlockSpec((HID, OUT), lambda j: (0, 0)),
        pl.BlockSpec((1, OUT), lambda j: (0, 0)),
    ]
    inputs = [h, W_up, b2d]
    kwargs = {}
    if out_prev is not None:
        in_specs.append(pl.BlockSpec(memory_space=pl.ANY))
        inputs.append(out_prev)
        kwargs["input_output_aliases"] = {3: 0}

    def mm_kernel(h_ref, w_ref, b_ref, *rest):
        o_ref = rest[-1]
        hb = jnp.maximum(h_ref[...], 0.0)
        o_ref[...] = (
            jnp.dot(hb, w_ref[...], preferred_element_type=jnp.float32)
            + b_ref[...]
        )

    return pl.pallas_call(
        mm_kernel,
        grid=(n_rows // _BLK,),
        in_specs=in_specs,
        out_specs=pl.BlockSpec((_BLK, OUT), lambda j: (i0 + j, 0)),
        out_shape=jax.ShapeDtypeStruct((n_total, OUT), jnp.float32),
        **kwargs,
    )(*inputs)


def kernel(x, emb_table, W_up, b_up):
    B, L = x.shape
    n = B * L
    x2d = x.reshape(n // _K, _K).astype(jnp.int32)
    b2d = b_up.reshape(1, OUT)

    rows_per_chunk = n // _NCHUNK
    idx_per_chunk = x2d.shape[0] // _NCHUNK

    hs = [
        _sc_gather(
            lax.slice_in_dim(x2d, i * idx_per_chunk, (i + 1) * idx_per_chunk),
            emb_table,
        )
        for i in range(_NCHUNK)
    ]
    out = None
    for i in range(_NCHUNK):
        out = _tc_project_into(hs[i], W_up, b2d, out, n, i * rows_per_chunk)
    return out.reshape(B, L, OUT)


# speedup vs baseline: 1.0123x; 1.0014x over previous
import functools

import jax
import jax.numpy as jnp
from jax import lax
from jax.experimental import pallas as pl
from jax.experimental.pallas import tpu as pltpu
from jax.experimental.pallas import tpu_sc as plsc

HID = 128
OUT = 512

_NC = 2
_NS = 16
_NW = _NC * _NS

_K = 128
_NB = 6
_NCHUNK = 1
_BLK = 10240


def _sc_gather(x2d, table):
    n_chunks = x2d.shape[0]
    chunks_per_w = n_chunks // _NW
    rows_per_w = chunks_per_w * _K
    n_groups = chunks_per_w // _NB
    rem = chunks_per_w - n_groups * _NB

    mesh = plsc.VectorSubcoreMesh(core_axis_name="c", subcore_axis_name="s")

    @functools.partial(
        pl.kernel,
        mesh=mesh,
        out_type=jax.ShapeDtypeStruct((n_chunks * _K, HID), jnp.float32),
        scratch_types=[
            pltpu.VMEM((chunks_per_w, _K), jnp.int32),
            pltpu.VMEM((_NB, _K, HID), jnp.float32),
        ] + [pltpu.SemaphoreType.DMA] * (2 * _NB),
    )
    def gather_kernel(idx_hbm, table_hbm, h_hbm, idx_v, rows_v, *sems):
        gsems = sems[:_NB]
        osems = sems[_NB:]
        wid = lax.axis_index("s") * _NC + lax.axis_index("c")
        chunk0 = wid * chunks_per_w
        row0 = wid * rows_per_w
        half = _NB // 2

        pltpu.sync_copy(idx_hbm.at[pl.ds(chunk0, chunks_per_w)], idx_v)

        def fire_gather(j, b):
            pltpu.async_copy(table_hbm.at[idx_v.at[j]], rows_v.at[b], gsems[b])

        def fire_wb(j, b):
            pltpu.async_copy(
                rows_v.at[b], h_hbm.at[pl.ds(row0 + j * _K, _K)], osems[b]
            )

        def wait_gather(b):
            pltpu.make_async_copy(
                table_hbm.at[idx_v.at[0]], rows_v.at[b], gsems[b]
            ).wait()

        def wait_wb(b):
            pltpu.make_async_copy(
                rows_v.at[b], h_hbm.at[pl.ds(row0, _K)], osems[b]
            ).wait()

        for b in range(half):
            fire_gather(b, b)

        def step(j, b, b2):
            wait_gather(b)
            fire_wb(j, b)

            @pl.when(j + half < chunks_per_w)
            def _():
                @pl.when(j >= half)
                def _():
                    wait_wb(b2)
                fire_gather(j + half, b2)

        def body(g, _):
            for i in range(_NB):
                step(g * _NB + i, i, (i + half) % _NB)
            return 0

        lax.fori_loop(0, n_groups, body, 0)
        for i in range(rem):
            step(n_groups * _NB + i, i, (i + half) % _NB)

        for b in range(min(_NB, chunks_per_w)):
            wait_wb(b)

    return gather_kernel(x2d, table)


def _tc_project_into(h, W_up, b2d, out_prev, n_total, row0):
    n_rows = h.shape[0]
    i0 = row0 // _BLK

    in_specs = [
        pl.BlockSpec((_BLK, HID), lambda j: (j, 0)),
        pl.BlockSpec((HID, OUT), lambda j: (0, 0)),
        pl.BlockSpec((1, OUT), lambda j: (0, 0)),
    ]
    inputs = [h, W_up, b2d]
    kwargs = {}
    if out_prev is not None:
        in_specs.append(pl.BlockSpec(memory_space=pl.ANY))
        inputs.append(out_prev)
        kwargs["input_output_aliases"] = {3: 0}

    def mm_kernel(h_ref, w_ref, b_ref, *rest):
        o_ref = rest[-1]
        hb = jnp.maximum(h_ref[...], 0.0)
        o_ref[...] = (
            jnp.dot(hb, w_ref[...], preferred_element_type=jnp.float32)
            + b_ref[...]
        )

    return pl.pallas_call(
        mm_kernel,
        grid=(n_rows // _BLK,),
        in_specs=in_specs,
        out_specs=pl.BlockSpec((_BLK, OUT), lambda j: (i0 + j, 0)),
        out_shape=jax.ShapeDtypeStruct((n_total, OUT), jnp.float32),
        **kwargs,
    )(*inputs)


def kernel(x, emb_table, W_up, b_up):
    B, L = x.shape
    n = B * L
    x2d = x.reshape(n // _K, _K).astype(jnp.int32)
    b2d = b_up.reshape(1, OUT)

    rows_per_chunk = n // _NCHUNK
    idx_per_chunk = x2d.shape[0] // _NCHUNK

    hs = [
        _sc_gather(
            lax.slice_in_dim(x2d, i * idx_per_chunk, (i + 1) * idx_per_chunk),
            emb_table,
        )
        for i in range(_NCHUNK)
    ]
    out = None
    for i in range(_NCHUNK):
        out = _tc_project_into(hs[i], W_up, b2d, out, n, i * rows_per_chunk)
    return out.reshape(B, L, OUT)
